# baseline (speedup 1.0000x reference)
import functools

import jax
import jax.numpy as jnp
from jax import lax
from jax.experimental import pallas as pl
from jax.experimental.pallas import tpu as pltpu
from jax.experimental.pallas import tpu_sc as plsc

T = 16384
DIM = 2048
NE = 64
TOPK = 8
NG = 8
TOPK_G = 4
SCALE = 2.5
GSZ = NE // NG

NEG = -1e30
L = 16
NW = 32
TW = T // NW
NB = TW // L


def _mm_body(x_ref, wt_ref, s_ref):
    s_ref[...] = jax.nn.sigmoid(
        jnp.dot(x_ref[...], wt_ref[...], preferred_element_type=jnp.float32))


@jax.jit
def _tc_scores(x, wt, rows=2048):
    return pl.pallas_call(
        _mm_body,
        grid=(T // rows,),
        in_specs=[
            pl.BlockSpec((rows, DIM), lambda i: (i, 0)),
            pl.BlockSpec((DIM, NE), lambda i: (0, 0)),
        ],
        out_specs=pl.BlockSpec((rows, NE), lambda i: (i, 0)),
        out_shape=jax.ShapeDtypeStruct((T, NE), jnp.float32),
    )(x, wt)


def _splat(v, dtype=jnp.int32):
    return jnp.full((L,), v, dtype)


def _sc_body(s_hbm, w_hbm, i_hbm, s_v, w_v, i_v):
    wid = lax.axis_index("s") * 2 + lax.axis_index("c")
    base = wid * TW
    pltpu.sync_copy(s_hbm.at[pl.ds(base * NE, TW * NE)], s_v)

    iota = lax.broadcasted_iota(jnp.int32, (L,), 0)

    def batch(b, carry):
        tokNE = (b * L + iota) * NE
        tokK = (b * L + iota) * TOPK

        gm = []
        for g in range(NG):
            vs = [plsc.load_gather(s_v, [tokNE + (g * GSZ + k)])
                  for k in range(GSZ)]
            while len(vs) > 1:
                vs = [jnp.maximum(vs[i], vs[i + 1])
                      for i in range(0, len(vs), 2)]
            gm.append(vs[0])

        gsel = []
        for _ in range(TOPK_G):
            t = gm
            while len(t) > 1:
                t = [jnp.maximum(t[i], t[i + 1]) for i in range(0, len(t), 2)]
            cur = t[0]
            gs = _splat(127)
            for g in range(NG - 1, -1, -1):
                gs = jnp.where(gm[g] == cur, g, gs)
            gsel.append(gs)
            gm = [jnp.where(gs == g, NEG, gm[g]) for g in range(NG)]

        a, b_, c, d = gsel
        lo0, hi0 = jnp.minimum(a, b_), jnp.maximum(a, b_)
        lo1, hi1 = jnp.minimum(c, d), jnp.maximum(c, d)
        g0 = jnp.minimum(lo0, lo1)
        t0 = jnp.maximum(lo0, lo1)
        g3 = jnp.maximum(hi0, hi1)
        t1 = jnp.minimum(hi0, hi1)
        g1 = jnp.minimum(t0, t1)
        g2 = jnp.maximum(t0, t1)
        bases = [g0 * GSZ, g1 * GSZ, g2 * GSZ, g3 * GSZ]

        ce = [bases[q] + k for q in range(TOPK_G) for k in range(GSZ)]
        cand = [plsc.load_gather(s_v, [tokNE + e]) for e in ce]

        wsum = None
        wouts, eouts = [], []
        for j in range(TOPK):
            tv, te = list(cand), list(ce)
            while len(tv) > 1:
                nv, ne_ = [], []
                for i in range(0, len(tv), 2):
                    better = tv[i + 1] > tv[i]
                    nv.append(jnp.where(better, tv[i + 1], tv[i]))
                    ne_.append(jnp.where(better, te[i + 1], te[i]))
                tv, te = nv, ne_
            cur, eb = tv[0], te[0]
            wouts.append(cur)
            eouts.append(eb)
            wsum = cur if wsum is None else wsum + cur
            cand = [jnp.where(ce[i] == eb, NEG, cand[i])
                    for i in range(len(cand))]

        inv = SCALE / wsum
        for j in range(TOPK):
            plsc.store_scatter(w_v, [tokK + j], wouts[j] * inv)
            plsc.store_scatter(i_v, [tokK + j], eouts[j])
        return carry

    lax.fori_loop(0, NB, batch, 0)

    pltpu.sync_copy(w_v, w_hbm.at[pl.ds(base * TOPK, TW * TOPK)])
    pltpu.sync_copy(i_v, i_hbm.at[pl.ds(base * TOPK, TW * TOPK)])


@jax.jit
def _sc_route(scores_flat):
    mesh = plsc.VectorSubcoreMesh(core_axis_name="c", subcore_axis_name="s")
    call = functools.partial(
        pl.kernel,
        mesh=mesh,
        compiler_params=pltpu.CompilerParams(use_tc_tiling_on_sc=False,
                                             needs_layout_passes=False),
        out_type=[
            jax.ShapeDtypeStruct((T * TOPK,), jnp.float32),
            jax.ShapeDtypeStruct((T * TOPK,), jnp.int32),
        ],
        scratch_types=[
            pltpu.VMEM((TW * NE,), jnp.float32),
            pltpu.VMEM((TW * TOPK,), jnp.float32),
            pltpu.VMEM((TW * TOPK,), jnp.int32),
        ],
    )(_sc_body)
    return call(scores_flat)


def kernel(x, weight):
    scores = _tc_scores(x, weight.T)
    w, i = _sc_route(scores.reshape(-1))
    return w.reshape(T, TOPK), i.reshape(T, TOPK)
